# baseline (speedup 1.0000x reference)
import functools

import jax
import jax.numpy as jnp
from jax import lax
from jax.experimental import pallas as pl
from jax.experimental.pallas import tpu as pltpu
from jax.experimental.pallas import tpu_sc as plsc

N = 16384
H = 768
EPS = 1e-05
POS_OFFSET = 2

NC, NS, L = 2, 16, 16
NW = NC * NS
TOK_PER_TILE = N // NW
C = 64
NCHUNK = TOK_PER_TILE // C
HV = H // L


def _allsum16(x):
    iota = lax.iota(jnp.int32, L)
    dnums = lax.GatherDimensionNumbers(
        offset_dims=(), collapsed_slice_dims=(0,), start_index_map=(0,))
    for k in (1, 2, 4, 8):
        idx = jnp.bitwise_xor(iota, k)
        x = x + lax.gather(x, idx[:, None], dnums, slice_sizes=(1,),
                           mode=lax.GatherScatterMode.PROMISE_IN_BOUNDS)
    return x


def _rsqrt16(x):
    i = lax.bitcast_convert_type(x, jnp.int32)
    i = jnp.int32(0x5F3759DF) - (i >> 1)
    y = lax.bitcast_convert_type(i, jnp.float32)
    for _ in range(3):
        y = y * (1.5 - 0.5 * x * y * y)
    return y


def _body(ids_hbm, pids_hbm, wt_hbm, pt_hbm, trow_hbm, g_hbm, b_hbm, out_hbm,
          widx, pidx, wrows, prows, trow, grow, brow, sem):
    wid = lax.axis_index("s") * NC + lax.axis_index("c")
    base = wid * TOK_PER_TILE

    pltpu.sync_copy(trow_hbm, trow)
    pltpu.sync_copy(g_hbm, grow)
    pltpu.sync_copy(b_hbm, brow)

    for chunk in range(NCHUNK):
        tok = pl.multiple_of(base + chunk * C, C)
        pltpu.sync_copy(ids_hbm.at[pl.ds(tok, C)], widx)
        pltpu.sync_copy(pids_hbm.at[pl.ds(tok, C)], pidx)
        for i in range(C // L):
            pidx[pl.ds(i * L, L)] = pidx[pl.ds(i * L, L)] + POS_OFFSET
        pltpu.async_copy(wt_hbm.at[widx], wrows, sem).wait()
        pltpu.async_copy(pt_hbm.at[pidx], prows, sem).wait()

        def token_body(t, _):
            def sum_body(j, carry):
                acc, acc2 = carry
                off = pl.multiple_of(j * L, L)
                v = (wrows[t, pl.ds(off, L)] + prows[t, pl.ds(off, L)]
                     + trow[pl.ds(off, L)])
                wrows[t, pl.ds(off, L)] = v
                return acc + v, acc2 + v * v

            zero = jnp.zeros((L,), jnp.float32)
            acc, acc2 = lax.fori_loop(0, HV, sum_body, (zero, zero))
            meanv = _allsum16(acc) * (1.0 / H)
            varv = _allsum16(acc2) * (1.0 / H) - meanv * meanv
            rstd = _rsqrt16(varv + EPS)

            def norm_body(j, _):
                off = pl.multiple_of(j * L, L)
                v = wrows[t, pl.ds(off, L)]
                wrows[t, pl.ds(off, L)] = ((v - meanv) * rstd
                                           * grow[pl.ds(off, L)]
                                           + brow[pl.ds(off, L)])
                return 0

            lax.fori_loop(0, HV, norm_body, 0)
            return 0

        lax.fori_loop(0, C, token_body, 0)
        pltpu.sync_copy(wrows, out_hbm.at[pl.ds(tok, C)])


_sc_call = functools.partial(
    pl.kernel,
    out_type=jax.ShapeDtypeStruct((N, H), jnp.float32),
    mesh=plsc.VectorSubcoreMesh(core_axis_name="c", subcore_axis_name="s"),
    scratch_types=[
        pltpu.VMEM((C,), jnp.int32),
        pltpu.VMEM((C,), jnp.int32),
        pltpu.VMEM((C, H), jnp.float32),
        pltpu.VMEM((C, H), jnp.float32),
        pltpu.VMEM((H,), jnp.float32),
        pltpu.VMEM((H,), jnp.float32),
        pltpu.VMEM((H,), jnp.float32),
        pltpu.SemaphoreType.DMA,
    ],
)(_body)


def kernel(input_ids, position_ids, word_emb, pos_emb, type_emb, ln_gamma,
           ln_beta):
    ids = input_ids.astype(jnp.int32)
    pids = position_ids.astype(jnp.int32)
    return _sc_call(ids, pids, word_emb, pos_emb, type_emb.reshape(H),
                    ln_gamma, ln_beta)
